# baseline (device time: 163390 ns/iter reference)
import jax
import jax.numpy as jnp
from jax import lax
from jax.experimental import pallas as pl
from jax.experimental.pallas import tpu as pltpu

N = 32
B, SQ, SKV, DM = 2, 512, 512, 768
HQ_PER, DH = 8, 64
FQ = HQ_PER * DH
ROWS = B * SQ
R = ROWS // N


def _body(x_ref, wq_ref, k_ref, v_ref, wo_ref, o_ref,
          q_s, ctx_s, p_s, scratch, send1, recv1, send2, recv2):
    me = lax.axis_index("i")

    bar = pltpu.get_barrier_semaphore()
    for k in range(1, N):
        j = lax.rem(me + k, N)
        pl.semaphore_signal(bar, inc=1, device_id=j,
                            device_id_type=pl.DeviceIdType.LOGICAL)

    xb = x_ref[:, :].astype(jnp.bfloat16)
    wqb = wq_ref[:, :].astype(jnp.bfloat16)
    q_s[:, :] = jnp.dot(xb, wqb,
                        preferred_element_type=jnp.float32).astype(jnp.bfloat16)

    qb = lax.broadcasted_iota(jnp.int32, (SQ, SKV), 0) // 64
    kb = lax.broadcasted_iota(jnp.int32, (SQ, SKV), 1) // 64
    mask = (qb == kb) | (kb == 0) | ((qb + kb) % 3 == 0)

    for b in range(B):
        for h in range(HQ_PER):
            qh = q_s[b * SQ:(b + 1) * SQ, h * DH:(h + 1) * DH]
            kh = k_ref[b, :, h * DH:(h + 1) * DH].astype(jnp.bfloat16)
            vh = v_ref[b, :, h * DH:(h + 1) * DH].astype(jnp.bfloat16)
            s = lax.dot_general(
                qh, kh, (((1,), (1,)), ((), ())),
                preferred_element_type=jnp.float32) * 0.125
            s = jnp.where(mask, s, -1e9)
            m = jnp.max(s, axis=-1, keepdims=True)
            w = jnp.exp(s - m)
            w = w / jnp.sum(w, axis=-1, keepdims=True)
            ctx = jnp.dot(w.astype(jnp.bfloat16), vh,
                          preferred_element_type=jnp.float32)
            ctx_s[b * SQ:(b + 1) * SQ, h * DH:(h + 1) * DH] = (
                ctx.astype(jnp.bfloat16))

    wob = wo_ref[:, :].astype(jnp.bfloat16)
    p_s[:, :] = jnp.dot(ctx_s[:, :], wob,
                        preferred_element_type=jnp.float32).astype(jnp.bfloat16)

    pl.semaphore_wait(bar, N - 1)

    sends = []

    for k in range(1, N):
        j = lax.rem(me + k, N)
        d = pltpu.make_async_remote_copy(
            src_ref=p_s.at[pl.ds(j * R, R), :],
            dst_ref=scratch.at[k - 1],
            send_sem=send1.at[k - 1],
            recv_sem=recv1.at[k - 1],
            device_id=j,
            device_id_type=pl.DeviceIdType.LOGICAL,
        )
        d.start()
        sends.append(d)

    acc = p_s[pl.ds(me * R, R), :].astype(jnp.float32)
    for k in range(1, N):
        w = pltpu.make_async_remote_copy(
            src_ref=p_s.at[pl.ds(0, R), :],
            dst_ref=scratch.at[k - 1],
            send_sem=send1.at[k - 1],
            recv_sem=recv1.at[k - 1],
            device_id=me,
            device_id_type=pl.DeviceIdType.LOGICAL,
        )
        w.wait_recv()
        acc = acc + scratch[k - 1].astype(jnp.float32)
    o_ref[pl.ds(me * R, R), :] = acc.astype(jnp.bfloat16)

    for k in range(1, N):
        j = lax.rem(me + k, N)
        d = pltpu.make_async_remote_copy(
            src_ref=o_ref.at[pl.ds(me * R, R), :],
            dst_ref=o_ref.at[pl.ds(me * R, R), :],
            send_sem=send2.at[k - 1],
            recv_sem=recv2.at[k - 1],
            device_id=j,
            device_id_type=pl.DeviceIdType.LOGICAL,
        )
        d.start()
        sends.append(d)

    for k in range(1, N):
        src_dev = lax.rem(me - k + N, N)
        w = pltpu.make_async_remote_copy(
            src_ref=o_ref.at[pl.ds(0, R), :],
            dst_ref=o_ref.at[pl.ds(src_dev * R, R), :],
            send_sem=send2.at[k - 1],
            recv_sem=recv2.at[k - 1],
            device_id=me,
            device_id_type=pl.DeviceIdType.LOGICAL,
        )
        w.wait_recv()

    for d in sends:
        d.wait_send()


def kernel(x, Wq, K_ext, V_ext, Wo):
    me = lax.axis_index("i")

    K = lax.dynamic_slice_in_dim(K_ext, me * HQ_PER, HQ_PER, axis=2)
    V = lax.dynamic_slice_in_dim(V_ext, me * HQ_PER, HQ_PER, axis=2)
    K2 = K.reshape(B, SKV, FQ)
    V2 = V.reshape(B, SKV, FQ)
    x2 = x.reshape(ROWS, DM)

    out = pl.pallas_call(
        _body,
        out_shape=jax.ShapeDtypeStruct((ROWS, DM), jnp.bfloat16),
        in_specs=[pl.BlockSpec(memory_space=pltpu.VMEM)] * 5,
        out_specs=pl.BlockSpec(memory_space=pltpu.VMEM),
        scratch_shapes=[
            pltpu.VMEM((ROWS, FQ), jnp.bfloat16),
            pltpu.VMEM((ROWS, FQ), jnp.bfloat16),
            pltpu.VMEM((ROWS, DM), jnp.bfloat16),
            pltpu.VMEM((N - 1, R, DM), jnp.bfloat16),
            pltpu.SemaphoreType.DMA((N - 1,)),
            pltpu.SemaphoreType.DMA((N - 1,)),
            pltpu.SemaphoreType.DMA((N - 1,)),
            pltpu.SemaphoreType.DMA((N - 1,)),
        ],
        compiler_params=pltpu.CompilerParams(collective_id=0),
    )(x2, Wq, K2, V2, Wo)

    return out.reshape(B, SQ, DM).astype(jnp.float32)
